# Initial kernel scaffold; baseline (speedup 1.0000x reference)
import functools

import jax
import jax.numpy as jnp
from jax import lax
from jax.experimental import pallas as pl
from jax.experimental.pallas import tpu as pltpu
from jax.experimental.pallas import tpu_sc as plsc

_N = 10000
_D = 128
_E = 320000
_NC = 2
_NS = 16
_L = 16
_GT = 158
_GC = _GT * _NS
_EP = _GC * 128
_RPT = _N // _NS

_mesh = plsc.VectorSubcoreMesh(
    core_axis_name="c", subcore_axis_name="s", num_cores=_NC, num_subcores=_NS
)


@functools.partial(
    pl.kernel,
    out_type=jax.ShapeDtypeStruct((2 * _N, _D), jnp.float32),
    mesh=_mesh,
    scratch_types=[
        pltpu.VMEM_SHARED((_N, _D), jnp.float32),
        pltpu.VMEM((_GT, 128), jnp.int32),
        pltpu.VMEM((_GT, 128), jnp.int32),
        pltpu.VMEM((_GT, 128), jnp.float32),
        pltpu.VMEM((2, 128, _D), jnp.float32),
        pltpu.SemaphoreType.DMA,
        pltpu.SemaphoreType.DMA,
    ],
)
def _spmm(emb, rows, cols, vals, out, acc, rows_v, cols_v, vals_v, gbuf,
          sem0, sem1):
    c = lax.axis_index("c")
    s = lax.axis_index("s")
    gbase = c * _GC + s * _GT

    pltpu.sync_copy(rows.at[pl.ds(gbase, _GT)], rows_v)
    pltpu.sync_copy(cols.at[pl.ds(gbase, _GT)], cols_v)
    pltpu.sync_copy(vals.at[pl.ds(gbase, _GT)], vals_v)

    z = jnp.zeros((_L,), jnp.float32)

    def zero_row(i, carry):
        for j in range(_D // _L):
            gbuf[0, i, pl.ds(j * _L, _L)] = z
        return carry

    lax.fori_loop(0, 128, zero_row, 0)
    for k in range(5):
        pltpu.sync_copy(gbuf.at[0, pl.ds(0, 125)],
                        acc.at[pl.ds(s * _RPT + k * 125, 125)])

    def gather(g, b, sem):
        pltpu.async_copy(emb.at[cols_v.at[g]], gbuf.at[b], sem)

    def gather_wait(g, b, sem):
        pltpu.make_async_copy(emb.at[cols_v.at[g]], gbuf.at[b], sem).wait()

    def scale(b, g):
        def eg_body(eg, carry):
            for i in range(_L):
                e = eg * _L + i
                vb = jnp.full((_L,), vals_v[g, e], jnp.float32)
                for j in range(_D // _L):
                    sl = pl.ds(j * _L, _L)
                    gbuf[b, e, sl] = gbuf[b, e, sl] * vb
            return carry

        lax.fori_loop(0, 128 // _L, eg_body, 0)

    def scatter_add(b, g):
        pltpu.sync_copy(gbuf.at[b], acc.at[rows_v.at[g]], add=True)

    gather(0, 0, sem0)
    plsc.subcore_barrier()

    def pair(p, carry):
        g0 = 2 * p
        g1 = g0 + 1
        gather(g1, 1, sem1)
        gather_wait(g0, 0, sem0)
        scale(0, g0)
        scatter_add(0, g0)

        @pl.when(p < _GT // 2 - 1)
        def _():
            gather(g0 + 2, 0, sem0)

        gather_wait(g1, 1, sem1)
        scale(1, g1)
        scatter_add(1, g1)
        return carry

    lax.fori_loop(0, _GT // 2, pair, 0)

    plsc.subcore_barrier()
    pltpu.sync_copy(acc.at[pl.ds(s * _RPT, _RPT)],
                    out.at[pl.ds(c * _N + s * _RPT, _RPT)])


def kernel(users_emb, items_emb, user_edge_index, user_edge_values,
           item_edge_index, item_edge_values):
    emb = jnp.concatenate([users_emb, items_emb], axis=0)
    pad = _EP - _E

    def prep(ei, ev, col_off):
        r = jnp.concatenate([ei[0].astype(jnp.int32),
                             jnp.zeros((pad,), jnp.int32)])
        cc = jnp.concatenate([ei[1].astype(jnp.int32) + col_off,
                              jnp.zeros((pad,), jnp.int32)])
        v = jnp.concatenate([ev, jnp.zeros((pad,), jnp.float32)])
        return r, cc, v

    ru, cu, vu = prep(user_edge_index, user_edge_values, 0)
    ri, ci, vi = prep(item_edge_index, item_edge_values, _N)
    rows = jnp.concatenate([ru, ri]).reshape(2 * _GC, 128)
    cols = jnp.concatenate([cu, ci]).reshape(2 * _GC, 128)
    vals = jnp.concatenate([vu, vi]).reshape(2 * _GC, 128)

    out = _spmm(emb, rows, cols, vals)
    return out[:_N], out[_N:]


# speedup vs baseline: 3.8125x; 3.8125x over previous
import functools

import jax
import jax.numpy as jnp
from jax import lax
from jax.experimental import pallas as pl
from jax.experimental.pallas import tpu as pltpu
from jax.experimental.pallas import tpu_sc as plsc

_N = 10000
_D = 128
_E = 320000
_NC = 2
_NS = 16
_L = 16
_GT = 160
_GC = _GT * _NS
_EP = _GC * 128
_SG = 16
_RPT = 640

_mesh = plsc.VectorSubcoreMesh(
    core_axis_name="c", subcore_axis_name="s", num_cores=_NC, num_subcores=_NS
)


@functools.partial(
    pl.kernel,
    out_type=jax.ShapeDtypeStruct((2 * _N, _D), jnp.float32),
    mesh=_mesh,
    scratch_types=[
        pltpu.VMEM_SHARED((_N, _D), jnp.float32),
        pltpu.VMEM((_SG, 128), jnp.int32),
        pltpu.VMEM((_SG, 128), jnp.int32),
        pltpu.VMEM((_SG, 128), jnp.float32),
        pltpu.VMEM((2, 128, _D), jnp.float32),
        pltpu.SemaphoreType.DMA,
        pltpu.SemaphoreType.DMA,
    ],
)
def _spmm(emb, rows, cols, vals, out, acc, rows_v, cols_v, vals_v, gbuf,
          sem0, sem1):
    c = lax.axis_index("c")
    s = lax.axis_index("s")
    gbase = c * _GC + s * _GT

    z = jnp.zeros((_L,), jnp.float32)

    def zero_row(i, carry):
        for j in range(_D // _L):
            gbuf[0, i, pl.ds(j * _L, _L)] = z
        return carry

    lax.fori_loop(0, 128, zero_row, 0)

    @pl.when(s < _NS - 1)
    def _():
        for k in range(_RPT // 128):
            pltpu.sync_copy(gbuf.at[0],
                            acc.at[pl.ds(s * _RPT + k * 128, 128)])

    @pl.when(s == _NS - 1)
    def _():
        base = (_NS - 1) * _RPT
        for k in range(3):
            pltpu.sync_copy(gbuf.at[0], acc.at[pl.ds(base + k * 128, 128)])
        pltpu.sync_copy(gbuf.at[0, pl.ds(0, 16)],
                        acc.at[pl.ds(base + 384, 16)])

    def gather(g, b, sem):
        pltpu.async_copy(emb.at[cols_v.at[g]], gbuf.at[b], sem)

    def gather_wait(g, b, sem):
        pltpu.make_async_copy(emb.at[cols_v.at[g]], gbuf.at[b], sem).wait()

    def scale(b, g):
        def eg_body(eg, carry):
            v16 = vals_v[g, pl.ds(eg * _L, _L)]
            for i in range(_L):
                e = eg * _L + i
                vb = jnp.full((_L,), v16[i], jnp.float32)
                for j in range(_D // _L):
                    sl = pl.ds(j * _L, _L)
                    gbuf[b, e, sl] = gbuf[b, e, sl] * vb
            return carry

        lax.fori_loop(0, 128 // _L, eg_body, 0)

    def scatter_add(b, g):
        pltpu.sync_copy(gbuf.at[b], acc.at[rows_v.at[g]], add=True)

    plsc.subcore_barrier()

    def super_chunk(sc, carry):
        gb = gbase + sc * _SG
        pltpu.sync_copy(rows.at[pl.ds(gb, _SG)], rows_v)
        pltpu.sync_copy(cols.at[pl.ds(gb, _SG)], cols_v)
        pltpu.sync_copy(vals.at[pl.ds(gb, _SG)], vals_v)

        gather(0, 0, sem0)
        def pair(p, carry2):
            g0 = 2 * p
            g1 = g0 + 1
            gather(g1, 1, sem1)
            gather_wait(g0, 0, sem0)
            scale(0, g0)
            scatter_add(0, g0)

            @pl.when(p < _SG // 2 - 1)
            def _():
                gather(g0 + 2, 0, sem0)

            gather_wait(g1, 1, sem1)
            scale(1, g1)
            scatter_add(1, g1)
            return carry2

        lax.fori_loop(0, _SG // 2, pair, 0)
        return carry

    lax.fori_loop(0, _GT // _SG, super_chunk, 0)

    plsc.subcore_barrier()

    @pl.when(s < _NS - 1)
    def _():
        pltpu.sync_copy(acc.at[pl.ds(s * _RPT, _RPT)],
                        out.at[pl.ds(c * _N + s * _RPT, _RPT)])

    @pl.when(s == _NS - 1)
    def _():
        base = (_NS - 1) * _RPT
        pltpu.sync_copy(acc.at[pl.ds(base, _N - base)],
                        out.at[pl.ds(c * _N + base, _N - base)])


def kernel(users_emb, items_emb, user_edge_index, user_edge_values,
           item_edge_index, item_edge_values):
    emb = jnp.concatenate([users_emb, items_emb], axis=0)
    pad = _EP - _E

    def prep(ei, ev, col_off):
        r = jnp.concatenate([ei[0].astype(jnp.int32),
                             jnp.zeros((pad,), jnp.int32)])
        cc = jnp.concatenate([ei[1].astype(jnp.int32) + col_off,
                              jnp.zeros((pad,), jnp.int32)])
        v = jnp.concatenate([ev, jnp.zeros((pad,), jnp.float32)])
        return r, cc, v

    ru, cu, vu = prep(user_edge_index, user_edge_values, 0)
    ri, ci, vi = prep(item_edge_index, item_edge_values, _N)
    rows = jnp.concatenate([ru, ri]).reshape(2 * _GC, 128)
    cols = jnp.concatenate([cu, ci]).reshape(2 * _GC, 128)
    vals = jnp.concatenate([vu, vi]).reshape(2 * _GC, 128)

    out = _spmm(emb, rows, cols, vals)
    return out[:_N], out[_N:]
